# baseline (speedup 1.0000x reference)
import functools

import jax
import jax.numpy as jnp
from jax import lax
from jax.experimental import pallas as pl
from jax.experimental.pallas import tpu as pltpu
from jax.experimental.pallas import tpu_sc as plsc

N = 16384
K = 26
G = 100
W = K * G

SC_ROWS = 4096
NUM_CORES = 2
NUM_SUBCORES = 16
NW = NUM_CORES * NUM_SUBCORES
ROWS_PER_W = SC_ROWS // NW
C = 16
CHUNKS = ROWS_PER_W // C
L = 16

TC_R = 512
TC_BLOCKS = (N - SC_ROWS) // TC_R


def _sc_worker(x_hbm, zeros_hbm, out_hbm, xv, buf0, buf1, pos0, pos1,
               sem0, sem1):
    wid = lax.axis_index("s") * NUM_CORES + lax.axis_index("c")
    base_row = wid * ROWS_PER_W

    pltpu.sync_copy(x_hbm.at[pl.ds(base_row * K, ROWS_PER_W * K)], xv)
    pltpu.sync_copy(zeros_hbm, buf0)
    pltpu.sync_copy(zeros_hbm, buf1)

    ci = lax.iota(jnp.int32, L)
    base_lo = ci * G
    base_hi = (ci + 10) * G
    ones_v = jnp.full((L,), 1.0, jnp.float32)
    zero_v = jnp.zeros((L,), jnp.float32)
    one_i = jnp.full((L,), 1, jnp.int32)

    def fill(g, buf, posb):
        for r in range(C):
            rl = g * C + r
            r_splat = jnp.full((L,), r, jnp.int32)
            for half, base_c in ((0, base_lo), (1, base_hi)):
                v = xv[pl.ds(rl * K + 10 * half, L)]
                cv = base_c + jnp.maximum(v, one_i) - 1
                plsc.store_scatter(buf, [r_splat, cv], ones_v, mask=v > 0)
                posb[pl.ds((2 * r + half) * L, L)] = cv

    def clear(buf, posb):
        for r in range(C):
            r_splat = jnp.full((L,), r, jnp.int32)
            for half in (0, 1):
                cv = posb[pl.ds((2 * r + half) * L, L)]
                plsc.store_scatter(buf, [r_splat, cv], zero_v)

    def dma_out(g, buf, sem):
        row0 = base_row + g * C
        return pltpu.make_async_copy(
            buf, out_hbm.at[pl.ds(row0, C), :], sem)

    fill(0, buf0, pos0)
    dma_out(0, buf0, sem0).start()
    fill(1, buf1, pos1)
    dma_out(1, buf1, sem1).start()

    def body(gg, _):
        for b, buf, posb, sem in ((0, buf0, pos0, sem0),
                                  (1, buf1, pos1, sem1)):
            g = 2 * gg + b
            dma_out(g - 2, buf, sem).wait()
            clear(buf, posb)
            fill(g, buf, posb)
            dma_out(g, buf, sem).start()
        return 0

    lax.fori_loop(1, CHUNKS // 2, body, 0)

    dma_out(CHUNKS - 2, buf0, sem0).wait()
    dma_out(CHUNKS - 1, buf1, sem1).wait()


_sc_call = functools.partial(
    pl.kernel,
    out_type=jax.ShapeDtypeStruct((SC_ROWS, W), jnp.float32),
    mesh=plsc.VectorSubcoreMesh(core_axis_name="c", subcore_axis_name="s",
                                num_cores=NUM_CORES,
                                num_subcores=NUM_SUBCORES),
    scratch_types=[
        pltpu.VMEM((ROWS_PER_W * K,), jnp.int32),
        pltpu.VMEM((C, W), jnp.float32),
        pltpu.VMEM((C, W), jnp.float32),
        pltpu.VMEM((C * 2 * L,), jnp.int32),
        pltpu.VMEM((C * 2 * L,), jnp.int32),
        pltpu.SemaphoreType.DMA,
        pltpu.SemaphoreType.DMA,
    ],
    compiler_params=pltpu.CompilerParams(use_tc_tiling_on_sc=True,
                                         needs_layout_passes=False),
)(_sc_worker)


def _tc_fill_body(x_ref, sel_ref, cls_ref, out_ref):
    xf = x_ref[...].astype(jnp.float32)
    xe = jnp.dot(xf, sel_ref[...], preferred_element_type=jnp.float32)
    out_ref[...] = (xe == cls_ref[...]).astype(jnp.float32)


def _tc_fill(x, sel, cls):
    base = SC_ROWS // TC_R
    return pl.pallas_call(
        _tc_fill_body,
        out_shape=jax.ShapeDtypeStruct((N, W), jnp.float32),
        grid=(TC_BLOCKS,),
        in_specs=[
            pl.BlockSpec((TC_R, K), lambda i: (base + i, 0)),
            pl.BlockSpec((K, W), lambda i: (0, 0)),
            pl.BlockSpec((1, W), lambda i: (0, 0)),
        ],
        out_specs=pl.BlockSpec((TC_R, W), lambda i: (base + i, 0)),
    )(x, sel, cls)


def _tc_stitch_body(sc_ref, filled_ref, out_ref):
    del filled_ref
    out_ref[...] = sc_ref[...]


def _tc_stitch(sc_out, filled):
    return pl.pallas_call(
        _tc_stitch_body,
        out_shape=jax.ShapeDtypeStruct((N, W), jnp.float32),
        grid=(SC_ROWS // TC_R,),
        in_specs=[
            pl.BlockSpec((TC_R, W), lambda i: (i, 0)),
            pl.BlockSpec(memory_space=pl.ANY),
        ],
        out_specs=pl.BlockSpec((TC_R, W), lambda i: (i, 0)),
        input_output_aliases={1: 0},
    )(sc_out, filled)


def kernel(x):
    xi = x.astype(jnp.int32)
    zeros = jnp.zeros((C, W), jnp.float32)
    sc_out = _sc_call(xi.reshape(-1), zeros)
    col = lax.broadcasted_iota(jnp.int32, (1, W), 1)
    comp = col // G
    sel = (comp == lax.broadcasted_iota(jnp.int32, (K, W), 0)
           ).astype(jnp.float32)
    cls = (col % G + 1).astype(jnp.float32)
    filled = _tc_fill(xi, sel, cls)
    return _tc_stitch(sc_out, filled)


# speedup vs baseline: 2.9567x; 2.9567x over previous
import functools

import jax
import jax.numpy as jnp
from jax import lax
from jax.experimental import pallas as pl
from jax.experimental.pallas import tpu as pltpu
from jax.experimental.pallas import tpu_sc as plsc

N = 16384
K = 26
G = 100
W = K * G

NUM_CORES = 2
NUM_SUBCORES = 16
NW = NUM_CORES * NUM_SUBCORES
COLS_PER_W = N // NW
PAIRS = K // 2
CB = 2 * G
CC = COLS_PER_W // 2
L = 16
GROUPS = CC // L


def _worker_body(xt_hbm, zeros_hbm, out_hbm, xv, buf0, buf1, pos0, pos1,
                 sem0, sem1):
    wid = lax.axis_index("s") * NUM_CORES + lax.axis_index("c")
    base_col = wid * COLS_PER_W

    pltpu.sync_copy(xt_hbm.at[:, pl.ds(base_col, COLS_PER_W)], xv)
    pltpu.sync_copy(zeros_hbm, buf0)
    pltpu.sync_copy(zeros_hbm, buf1)

    ci = lax.iota(jnp.int32, L)
    ones_v = jnp.full((L,), 1.0, jnp.float32)
    zero_v = jnp.zeros((L,), jnp.float32)
    one_i = jnp.full((L,), 1, jnp.int32)

    def fill(p, h, buf, posb):
        for half in (0, 1):
            base_r = half * G
            for g in range(GROUPS):
                cols = ci + g * L
                v = xv[2 * p + half, pl.ds(h * CC + g * L, L)]
                rv = base_r + jnp.maximum(v, one_i) - 1
                plsc.store_scatter(buf, [rv, cols], ones_v, mask=v > 0)
                posb[pl.ds((half * GROUPS + g) * L, L)] = rv

    def clear(buf, posb):
        for half in (0, 1):
            for g in range(GROUPS):
                cols = ci + g * L
                rv = posb[pl.ds((half * GROUPS + g) * L, L)]
                plsc.store_scatter(buf, [rv, cols], zero_v)

    def dma_out(p, h, buf, sem):
        return pltpu.make_async_copy(
            buf,
            out_hbm.at[pl.ds(p * CB, CB), pl.ds(base_col + h * CC, CC)],
            sem)

    fill(0, 0, buf0, pos0)
    dma_out(0, 0, buf0, sem0).start()
    fill(0, 1, buf1, pos1)
    dma_out(0, 1, buf1, sem1).start()

    def body(p, _):
        for h, buf, posb, sem in ((0, buf0, pos0, sem0),
                                  (1, buf1, pos1, sem1)):
            dma_out(p - 1, h, buf, sem).wait()
            clear(buf, posb)
            fill(p, h, buf, posb)
            dma_out(p, h, buf, sem).start()
        return 0

    lax.fori_loop(1, PAIRS, body, 0)

    dma_out(PAIRS - 1, 0, buf0, sem0).wait()
    dma_out(PAIRS - 1, 1, buf1, sem1).wait()


_sc_call = functools.partial(
    pl.kernel,
    out_type=jax.ShapeDtypeStruct((W, N), jnp.float32),
    mesh=plsc.VectorSubcoreMesh(core_axis_name="c", subcore_axis_name="s",
                                num_cores=NUM_CORES,
                                num_subcores=NUM_SUBCORES),
    scratch_types=[
        pltpu.VMEM((K, COLS_PER_W), jnp.int32),
        pltpu.VMEM((CB, CC), jnp.float32),
        pltpu.VMEM((CB, CC), jnp.float32),
        pltpu.VMEM((2 * GROUPS * L,), jnp.int32),
        pltpu.VMEM((2 * GROUPS * L,), jnp.int32),
        pltpu.SemaphoreType.DMA,
        pltpu.SemaphoreType.DMA,
    ],
    compiler_params=pltpu.CompilerParams(use_tc_tiling_on_sc=True,
                                         needs_layout_passes=False),
)(_worker_body)


def kernel(x):
    xt = jnp.swapaxes(x.astype(jnp.int32), 0, 1)
    zeros = jnp.zeros((CB, CC), jnp.float32)
    out_t = _sc_call(xt, zeros)
    return jnp.swapaxes(out_t, 0, 1)
